# baseline (speedup 1.0000x reference)
import functools

import jax
import jax.numpy as jnp
from jax import lax
from jax.experimental import pallas as pl
from jax.experimental.pallas import tpu as pltpu
from jax.experimental.pallas import tpu_sc as plsc

_B = 16384
_D = 32
_NROWS = 1000000
_NC = 2
_NS = 16
_NW = _NC * _NS
_RPT = _B // _NW
_HALF = _RPT // 2
_L = 16
_TAIL = (_NROWS // 128) * 128
_PACKED = _TAIL // 4
_KCOLS = 512
_GRID = _TAIL // _KCOLS


def _tc_pack(uwt, iwt):
    def body(u_ref, i_ref, up_ref, ip_ref):
        for src, dst in ((u_ref, up_ref), (i_ref, ip_ref)):
            x = src[...]
            y = x.reshape(_D, _KCOLS // 4, 4).transpose(1, 2, 0)
            dst[...] = y.reshape(_KCOLS // 4, 128)

    return pl.pallas_call(
        body,
        grid=(_GRID,),
        in_specs=[
            pl.BlockSpec((_D, _KCOLS), lambda i: (0, i)),
            pl.BlockSpec((_D, _KCOLS), lambda i: (0, i)),
        ],
        out_specs=[
            pl.BlockSpec((_KCOLS // 4, 128), lambda i: (i, 0)),
            pl.BlockSpec((_KCOLS // 4, 128), lambda i: (i, 0)),
        ],
        out_shape=[
            jax.ShapeDtypeStruct((_PACKED, 128), jnp.float32),
            jax.ShapeDtypeStruct((_PACKED, 128), jnp.float32),
        ],
    )(uwt, iwt)


@functools.lru_cache(maxsize=1)
def _sc_diff_kernel():
    mesh = plsc.VectorSubcoreMesh(core_axis_name="c", subcore_axis_name="s")

    @functools.partial(
        pl.kernel,
        mesh=mesh,
        compiler_params=pltpu.CompilerParams(
            needs_layout_passes=False, use_tc_tiling_on_sc=False
        ),
        out_type=jax.ShapeDtypeStruct((_B,), jnp.float32),
        scratch_types=[
            pltpu.VMEM((_RPT,), jnp.int32),
            pltpu.VMEM((_RPT,), jnp.int32),
            pltpu.VMEM((_RPT,), jnp.int32),
            pltpu.VMEM((_RPT,), jnp.int32),
            pltpu.VMEM((_RPT,), jnp.int32),
            pltpu.VMEM((_RPT,), jnp.int32),
            pltpu.VMEM((_HALF, 128), jnp.float32),
            pltpu.VMEM((_HALF, 128), jnp.float32),
            pltpu.VMEM((_HALF, 128), jnp.float32),
            pltpu.VMEM((64, _D), jnp.float32),
            pltpu.VMEM((64, _D), jnp.float32),
            pltpu.VMEM((_RPT,), jnp.float32),
            pltpu.SemaphoreType.DMA,
        ],
    )
    def diff_kernel(users_hbm, items_hbm, negs_hbm, up_hbm, ip_hbm,
                    tail_u_hbm, tail_i_hbm, out_hbm,
                    u_idx, p_idx, n_idx, u4, p4, n4,
                    ur_v, pr_v, nr_v, tu_v, ti_v, out_v, sem):
        wid = lax.axis_index("s") * _NC + lax.axis_index("c")
        base = wid * _RPT
        pltpu.sync_copy(users_hbm.at[pl.ds(base, _RPT)], u_idx)
        pltpu.sync_copy(items_hbm.at[pl.ds(base, _RPT)], p_idx)
        pltpu.sync_copy(negs_hbm.at[pl.ds(base, _RPT)], n_idx)
        pltpu.sync_copy(tail_u_hbm, tu_v)
        pltpu.sync_copy(tail_i_hbm, ti_v)

        def pack_idx(g, carry):
            j0 = g * _L
            sl = pl.ds(j0, _L)
            u4[sl] = jnp.minimum(u_idx[sl], _TAIL - 1) >> 2
            p4[sl] = jnp.minimum(p_idx[sl], _TAIL - 1) >> 2
            n4[sl] = jnp.minimum(n_idx[sl], _TAIL - 1) >> 2
            return carry

        lax.fori_loop(0, _RPT // _L, pack_idx, 0)

        lane = lax.iota(jnp.int32, _L)
        perms = [lane ^ k for k in (8, 4, 2, 1)]

        def merge(a, b, kk):
            k, pk = kk
            return jnp.where((lane & k) == 0, a + a[pk], b + b[pk])

        for h in range(2):
            hsl = pl.ds(h * _HALF, _HALF)
            cu = pltpu.async_copy(up_hbm.at[u4.at[hsl]], ur_v, sem)
            cp = pltpu.async_copy(ip_hbm.at[p4.at[hsl]], pr_v, sem)
            cn = pltpu.async_copy(ip_hbm.at[n4.at[hsl]], nr_v, sem)
            cu.wait()
            cp.wait()
            cn.wait()

            def blk_body(bb, carry):
                j0 = h * _HALF + bb * _L
                ru = u_idx[pl.ds(j0, _L)]
                rp = p_idx[pl.ds(j0, _L)]
                rn = n_idx[pl.ds(j0, _L)]
                s = []
                for i in range(_L):
                    jl = bb * _L + i

                    def fetch(rvec, rows_v, tail_v):
                        r = rvec[i]
                        m = (r & 3) << 5
                        a = rows_v[jl, pl.ds(m, _L)]
                        b = rows_v[jl, pl.ds(m + _L, _L)]
                        rt = jnp.clip(r - _TAIL, 0, 63)
                        ta = tail_v[rt, pl.ds(0, _L)]
                        tb = tail_v[rt, pl.ds(_L, _L)]
                        is_tail = r >= _TAIL
                        return (jnp.where(is_tail, ta, a),
                                jnp.where(is_tail, tb, b))

                    u1, u2 = fetch(ru, ur_v, tu_v)
                    p1, p2 = fetch(rp, pr_v, ti_v)
                    n1, n2 = fetch(rn, nr_v, ti_v)
                    s.append(u1 * (p1 - n1) + u2 * (p2 - n2))
                for lev, k in enumerate((8, 4, 2, 1)):
                    kk = (k, perms[lev])
                    s = [merge(s[2 * q], s[2 * q + 1], kk)
                         for q in range(len(s) // 2)]
                rows = j0 + lane
                plsc.store_scatter(out_v, [rows], s[0])
                return carry

            lax.fori_loop(0, _HALF // _L, blk_body, 0)

        pltpu.sync_copy(out_v, out_hbm.at[pl.ds(base, _RPT)])

    return diff_kernel


def _tc_loss(diff2d):
    def body(x_ref, o_ref):
        x = x_ref[...]
        ls = jnp.minimum(x, 0.0) - jnp.log1p(jnp.exp(-jnp.abs(x)))
        o_ref[0, 0] = -(jnp.sum(ls) / _B)

    return pl.pallas_call(
        body,
        out_shape=jax.ShapeDtypeStruct((1, 1), jnp.float32),
        out_specs=pl.BlockSpec(memory_space=pltpu.SMEM),
    )(diff2d)


def kernel(batch, neg_items, users_weight, items_weight):
    users = batch[:, 0].astype(jnp.int32)
    items = batch[:, 2].astype(jnp.int32)
    negs = neg_items.astype(jnp.int32)
    uwt = users_weight.T
    iwt = items_weight.T
    tail_u = users_weight[_TAIL:, :]
    tail_i = items_weight[_TAIL:, :]
    up, ip = _tc_pack(uwt, iwt)
    diff = _sc_diff_kernel()(users, items, negs, up, ip, tail_u, tail_i)
    loss = _tc_loss(diff.reshape(128, 128))
    return loss[0, 0]


# speedup vs baseline: 4.1079x; 4.1079x over previous
import functools

import jax
import jax.numpy as jnp
from jax import lax
from jax.experimental import pallas as pl
from jax.experimental.pallas import tpu as pltpu
from jax.experimental.pallas import tpu_sc as plsc

_B = 16384
_D = 32
_NROWS = 1000000
_NC = 2
_NS = 16
_NW = _NC * _NS
_RPT = _B // _NW
_HALF = _RPT // 2
_L = 16
_TAIL = (_NROWS // 128) * 128
_PACKED = _TAIL // 4
_KCOLS = 512
_GRID = _TAIL // _KCOLS


def _tc_pack(uwt, iwt):
    def body(u_ref, i_ref, up_ref, ip_ref):
        for src, dst in ((u_ref, up_ref), (i_ref, ip_ref)):
            x = src[...]
            for m in range(4):
                dst[:, pl.ds(_D * m, _D)] = x[:, 128 * m:128 * (m + 1)].T

    return pl.pallas_call(
        body,
        grid=(_GRID,),
        in_specs=[
            pl.BlockSpec((_D, _KCOLS), lambda i: (0, i)),
            pl.BlockSpec((_D, _KCOLS), lambda i: (0, i)),
        ],
        out_specs=[
            pl.BlockSpec((_KCOLS // 4, 128), lambda i: (i, 0)),
            pl.BlockSpec((_KCOLS // 4, 128), lambda i: (i, 0)),
        ],
        out_shape=[
            jax.ShapeDtypeStruct((_PACKED, 128), jnp.float32),
            jax.ShapeDtypeStruct((_PACKED, 128), jnp.float32),
        ],
    )(uwt, iwt)


@functools.lru_cache(maxsize=1)
def _sc_diff_kernel():
    mesh = plsc.VectorSubcoreMesh(core_axis_name="c", subcore_axis_name="s")

    @functools.partial(
        pl.kernel,
        mesh=mesh,
        compiler_params=pltpu.CompilerParams(
            needs_layout_passes=False, use_tc_tiling_on_sc=False
        ),
        out_type=jax.ShapeDtypeStruct((_B,), jnp.float32),
        scratch_types=[
            pltpu.VMEM((_RPT,), jnp.int32),
            pltpu.VMEM((_RPT,), jnp.int32),
            pltpu.VMEM((_RPT,), jnp.int32),
            pltpu.VMEM((_RPT,), jnp.int32),
            pltpu.VMEM((_RPT,), jnp.int32),
            pltpu.VMEM((_RPT,), jnp.int32),
            pltpu.VMEM((_HALF, 128), jnp.float32),
            pltpu.VMEM((_HALF, 128), jnp.float32),
            pltpu.VMEM((_HALF, 128), jnp.float32),
            pltpu.VMEM((64, _D), jnp.float32),
            pltpu.VMEM((64, _D), jnp.float32),
            pltpu.VMEM((_RPT,), jnp.float32),
            pltpu.SemaphoreType.DMA,
        ],
    )
    def diff_kernel(users_hbm, items_hbm, negs_hbm, up_hbm, ip_hbm,
                    tail_u_hbm, tail_i_hbm, out_hbm,
                    u_idx, p_idx, n_idx, u4, p4, n4,
                    ur_v, pr_v, nr_v, tu_v, ti_v, out_v, sem):
        wid = lax.axis_index("s") * _NC + lax.axis_index("c")
        base = wid * _RPT
        pltpu.sync_copy(users_hbm.at[pl.ds(base, _RPT)], u_idx)
        pltpu.sync_copy(items_hbm.at[pl.ds(base, _RPT)], p_idx)
        pltpu.sync_copy(negs_hbm.at[pl.ds(base, _RPT)], n_idx)
        pltpu.sync_copy(tail_u_hbm, tu_v)
        pltpu.sync_copy(tail_i_hbm, ti_v)

        def packed_row(r):
            return ((r >> 9) << 7) + (r & 127)

        def pack_idx(g, carry):
            j0 = g * _L
            sl = pl.ds(j0, _L)
            u4[sl] = packed_row(jnp.minimum(u_idx[sl], _TAIL - 1))
            p4[sl] = packed_row(jnp.minimum(p_idx[sl], _TAIL - 1))
            n4[sl] = packed_row(jnp.minimum(n_idx[sl], _TAIL - 1))
            return carry

        lax.fori_loop(0, _RPT // _L, pack_idx, 0)

        lane = lax.iota(jnp.int32, _L)
        perms = [lane ^ k for k in (8, 4, 2, 1)]

        def merge(a, b, kk):
            k, pk = kk
            return jnp.where((lane & k) == 0, a + a[pk], b + b[pk])

        for h in range(2):
            hsl = pl.ds(h * _HALF, _HALF)
            cu = pltpu.async_copy(up_hbm.at[u4.at[hsl]], ur_v, sem)
            cp = pltpu.async_copy(ip_hbm.at[p4.at[hsl]], pr_v, sem)
            cn = pltpu.async_copy(ip_hbm.at[n4.at[hsl]], nr_v, sem)
            cu.wait()
            cp.wait()
            cn.wait()

            def blk_body(bb, carry):
                j0 = h * _HALF + bb * _L
                ru = u_idx[pl.ds(j0, _L)]
                rp = p_idx[pl.ds(j0, _L)]
                rn = n_idx[pl.ds(j0, _L)]
                s = []
                for i in range(_L):
                    jl = bb * _L + i

                    def fetch(rvec, rows_v, tail_v):
                        r = rvec[i]
                        m = ((r >> 7) & 3) << 5
                        a = rows_v[jl, pl.ds(m, _L)]
                        b = rows_v[jl, pl.ds(m + _L, _L)]
                        rt = jnp.clip(r - _TAIL, 0, 63)
                        ta = tail_v[rt, pl.ds(0, _L)]
                        tb = tail_v[rt, pl.ds(_L, _L)]
                        is_tail = r >= _TAIL
                        return (jnp.where(is_tail, ta, a),
                                jnp.where(is_tail, tb, b))

                    u1, u2 = fetch(ru, ur_v, tu_v)
                    p1, p2 = fetch(rp, pr_v, ti_v)
                    n1, n2 = fetch(rn, nr_v, ti_v)
                    s.append(u1 * (p1 - n1) + u2 * (p2 - n2))
                for lev, k in enumerate((8, 4, 2, 1)):
                    kk = (k, perms[lev])
                    s = [merge(s[2 * q], s[2 * q + 1], kk)
                         for q in range(len(s) // 2)]
                rows = j0 + lane
                plsc.store_scatter(out_v, [rows], s[0])
                return carry

            lax.fori_loop(0, _HALF // _L, blk_body, 0)

        pltpu.sync_copy(out_v, out_hbm.at[pl.ds(base, _RPT)])

    return diff_kernel


def _tc_loss(diff2d):
    def body(x_ref, o_ref):
        x = x_ref[...]
        ls = jnp.minimum(x, 0.0) - jnp.log1p(jnp.exp(-jnp.abs(x)))
        o_ref[0, 0] = -(jnp.sum(ls) / _B)

    return pl.pallas_call(
        body,
        out_shape=jax.ShapeDtypeStruct((1, 1), jnp.float32),
        out_specs=pl.BlockSpec(memory_space=pltpu.SMEM),
    )(diff2d)


def kernel(batch, neg_items, users_weight, items_weight):
    users = batch[:, 0].astype(jnp.int32)
    items = batch[:, 2].astype(jnp.int32)
    negs = neg_items.astype(jnp.int32)
    uwt = users_weight.T
    iwt = items_weight.T
    tail_u = users_weight[_TAIL:, :]
    tail_i = items_weight[_TAIL:, :]
    up, ip = _tc_pack(uwt, iwt)
    diff = _sc_diff_kernel()(users, items, negs, up, ip, tail_u, tail_i)
    loss = _tc_loss(diff.reshape(128, 128))
    return loss[0, 0]


# speedup vs baseline: 5.9607x; 1.4510x over previous
import functools

import jax
import jax.numpy as jnp
from jax import lax
from jax.experimental import pallas as pl
from jax.experimental.pallas import tpu as pltpu
from jax.experimental.pallas import tpu_sc as plsc

_B = 16384
_D = 32
_NC = 2
_NS = 16
_NW = _NC * _NS
_RPT = _B // _NW
_L = 16


@functools.lru_cache(maxsize=1)
def _sc_diff_kernel():
    mesh = plsc.VectorSubcoreMesh(core_axis_name="c", subcore_axis_name="s")

    @functools.partial(
        pl.kernel,
        mesh=mesh,
        compiler_params=pltpu.CompilerParams(
            needs_layout_passes=False, use_tc_tiling_on_sc=False
        ),
        out_type=jax.ShapeDtypeStruct((_B,), jnp.float32),
        scratch_types=[
            pltpu.VMEM((_RPT,), jnp.int32),
            pltpu.VMEM((_RPT,), jnp.int32),
            pltpu.VMEM((_RPT,), jnp.int32),
            pltpu.VMEM((_RPT, _D), jnp.float32),
            pltpu.VMEM((_RPT, _D), jnp.float32),
            pltpu.VMEM((_RPT, _D), jnp.float32),
            pltpu.VMEM((_RPT,), jnp.float32),
            pltpu.SemaphoreType.DMA,
        ],
    )
    def diff_kernel(users_hbm, items_hbm, negs_hbm, uw_hbm, iw_hbm, out_hbm,
                    u_idx, p_idx, n_idx, u_rows, p_rows, n_rows, out_v, sem):
        wid = lax.axis_index("s") * _NC + lax.axis_index("c")
        base = wid * _RPT
        pltpu.sync_copy(users_hbm.at[pl.ds(base, _RPT)], u_idx)
        pltpu.sync_copy(items_hbm.at[pl.ds(base, _RPT)], p_idx)
        pltpu.sync_copy(negs_hbm.at[pl.ds(base, _RPT)], n_idx)
        cu = pltpu.async_copy(uw_hbm.at[u_idx], u_rows, sem)
        cp = pltpu.async_copy(iw_hbm.at[p_idx], p_rows, sem)
        cn = pltpu.async_copy(iw_hbm.at[n_idx], n_rows, sem)
        cu.wait()
        cp.wait()
        cn.wait()

        def blk_body(blk, carry):
            rows = blk * _L + lax.iota(jnp.int32, _L)
            acc = jnp.zeros((_L,), jnp.float32)
            for d in range(_D):
                dd = jnp.full((_L,), d, jnp.int32)
                u = plsc.load_gather(u_rows, [rows, dd])
                p = plsc.load_gather(p_rows, [rows, dd])
                n = plsc.load_gather(n_rows, [rows, dd])
                acc = acc + u * (p - n)
            plsc.store_scatter(out_v, [rows], acc)
            return carry

        lax.fori_loop(0, _RPT // _L, blk_body, 0)
        pltpu.sync_copy(out_v, out_hbm.at[pl.ds(base, _RPT)])

    return diff_kernel


def _tc_loss(diff2d):
    def body(x_ref, o_ref):
        x = x_ref[...]
        ls = jnp.minimum(x, 0.0) - jnp.log1p(jnp.exp(-jnp.abs(x)))
        o_ref[0, 0] = -(jnp.sum(ls) / _B)

    return pl.pallas_call(
        body,
        out_shape=jax.ShapeDtypeStruct((1, 1), jnp.float32),
        out_specs=pl.BlockSpec(memory_space=pltpu.SMEM),
    )(diff2d)


def kernel(batch, neg_items, users_weight, items_weight):
    users = batch[:, 0].astype(jnp.int32)
    items = batch[:, 2].astype(jnp.int32)
    negs = neg_items.astype(jnp.int32)
    diff = _sc_diff_kernel()(users, items, negs, users_weight, items_weight)
    loss = _tc_loss(diff.reshape(128, 128))
    return loss[0, 0]
